# baseline (speedup 1.0000x reference)
import functools

import jax
import jax.numpy as jnp
from jax import lax
from jax.experimental import pallas as pl
from jax.experimental.pallas import tpu as pltpu
from jax.experimental.pallas import tpu_sc as plsc

_BS = 512
_E = 64
_NW = 32
_L = 16


def _logits_blk(x_ref, w_ref, out_ref):
    out_ref[:] = jax.lax.dot_general(
        w_ref[:], x_ref[:],
        (((1,), (1,)), ((), ())),
        preferred_element_type=jnp.float32,
    )


def _sc_gating(logits_hbm, gates_hbm, idx_hbm, logits_v, gates_v, idx_v):
    n_tok = logits_hbm.shape[1]
    chunk = n_tok // _NW
    wid = lax.axis_index("s") * 2 + lax.axis_index("c")
    base = wid * chunk
    pltpu.sync_copy(logits_hbm.at[:, pl.ds(base, chunk)], logits_v)

    n_grp = chunk // _L
    for g in range(n_grp):
        off = g * _L

        def amax_body(e, carry):
            m, bi = carry
            v = logits_v[e, pl.ds(off, _L)]
            upd = v > m
            bi = jnp.where(upd, jnp.full((_L,), e, jnp.int32), bi)
            m = jnp.maximum(m, v)
            return m, bi

        m0 = jnp.full((_L,), -jnp.inf, jnp.float32)
        b0 = jnp.zeros((_L,), jnp.int32)
        m, bi = lax.fori_loop(0, _E, amax_body, (m0, b0))

        def sum_body(e, s):
            v = logits_v[e, pl.ds(off, _L)]
            return s + jnp.exp(v - m)

        s = lax.fori_loop(0, _E, sum_body, jnp.zeros((_L,), jnp.float32))
        gates_v[pl.ds(off, _L)] = 1.0 / s
        idx_v[pl.ds(off, _L)] = bi

    pltpu.sync_copy(gates_v, gates_hbm.at[pl.ds(base, chunk)])
    pltpu.sync_copy(idx_v, idx_hbm.at[pl.ds(base, chunk)])


@jax.jit
def kernel(x, complexity, W_router, cg_w, cg_b):
    B, S, D = x.shape
    E = W_router.shape[0]
    n = (B * S) // _BS
    x2 = x.reshape(B * S, D)
    logits_t = pl.pallas_call(
        _logits_blk,
        grid=(n,),
        in_specs=[
            pl.BlockSpec((_BS, D), lambda i: (i, 0)),
            pl.BlockSpec((E, D), lambda i: (0, 0)),
        ],
        out_specs=pl.BlockSpec((E, _BS), lambda i: (0, i)),
        out_shape=jax.ShapeDtypeStruct((E, B * S), jnp.float32),
        compiler_params=pltpu.CompilerParams(
            dimension_semantics=("arbitrary",),
        ),
    )(x2, W_router)

    n_tok = B * S
    chunk = n_tok // _NW
    sc = functools.partial(
        pl.kernel,
        mesh=plsc.VectorSubcoreMesh(core_axis_name="c", subcore_axis_name="s"),
        out_type=[
            jax.ShapeDtypeStruct((n_tok,), jnp.float32),
            jax.ShapeDtypeStruct((n_tok,), jnp.int32),
        ],
        scratch_types=[
            pltpu.VMEM((_E, chunk), jnp.float32),
            pltpu.VMEM((chunk,), jnp.float32),
            pltpu.VMEM((chunk,), jnp.int32),
        ],
    )(_sc_gating)
    gates, idx = sc(logits_t)
    return gates.reshape(B, S), idx.reshape(B, S)


# speedup vs baseline: 1.5594x; 1.5594x over previous
import functools

import jax
import jax.numpy as jnp
from jax.experimental import pallas as pl
from jax.experimental.pallas import tpu as pltpu

_BS = 512


def _router_blk(x_ref, w_ref, gates_ref, idx_ref):
    logits = jax.lax.dot_general(
        w_ref[:], x_ref[:],
        (((1,), (1,)), ((), ())),
        preferred_element_type=jnp.float32,
    )
    m = jnp.max(logits, axis=0)
    s = jnp.sum(jnp.exp(logits - m[None, :]), axis=0)
    gates_ref[0, 0, :] = 1.0 / s
    idx_ref[0, 0, :] = jnp.argmax(logits, axis=0).astype(jnp.int32)


@functools.partial(jax.jit, static_argnames=())
def kernel(x, complexity, W_router, cg_w, cg_b):
    B, S, D = x.shape
    E = W_router.shape[0]
    n = (B * S) // _BS
    x2 = x.reshape(B * S, D)
    gates, idx = pl.pallas_call(
        _router_blk,
        grid=(n,),
        in_specs=[
            pl.BlockSpec((_BS, D), lambda i: (i, 0)),
            pl.BlockSpec((E, D), lambda i: (0, 0)),
        ],
        out_specs=[
            pl.BlockSpec((1, 1, _BS), lambda i: (i, 0, 0)),
            pl.BlockSpec((1, 1, _BS), lambda i: (i, 0, 0)),
        ],
        out_shape=[
            jax.ShapeDtypeStruct((n, 1, _BS), jnp.float32),
            jax.ShapeDtypeStruct((n, 1, _BS), jnp.int32),
        ],
        compiler_params=pltpu.CompilerParams(
            dimension_semantics=("parallel",),
        ),
    )(x2, W_router)
    return gates.reshape(B, S), idx.reshape(B, S)
